# baseline (speedup 1.0000x reference)
import functools

import jax
import jax.numpy as jnp
from jax import lax
from jax.experimental import pallas as pl
from jax.experimental.pallas import tpu as pltpu
from jax.experimental.pallas import tpu_sc as plsc

NS = 16
NC = 2
NW = NC * NS
CH = 128
BLK = 512
DEGW = 16
HD = 64
UNIT = 2


def _leaky(v):
    return jnp.where(v > 0, v, 0.1 * v)


@functools.cache
def _build_deg_kernel(n_pad, nch):
    rows = n_pad // NS
    mesh = plsc.VectorSubcoreMesh(core_axis_name="c", subcore_axis_name="s")

    def body(row_a, row_b, ones_h, zer_h, out_h, idx_v, ones_v, da_sp, db_sp):
        c = lax.axis_index("c")
        s = lax.axis_index("s")
        wid = c * NS + s
        base = s * rows
        pltpu.sync_copy(zer_h, da_sp.at[pl.ds(base, rows)])
        pltpu.sync_copy(zer_h, db_sp.at[pl.ds(base, rows)])
        pltpu.sync_copy(ones_h, ones_v)
        pltpu.sync_copy(row_a.at[wid], idx_v)
        plsc.subcore_barrier()

        @pl.loop(0, nch)
        def _(j):
            pltpu.sync_copy(ones_v, da_sp.at[idx_v.at[j]], add=True)

        pltpu.sync_copy(row_b.at[wid], idx_v)

        @pl.loop(0, nch)
        def _(j):
            pltpu.sync_copy(ones_v, db_sp.at[idx_v.at[j]], add=True)

        plsc.subcore_barrier()
        pltpu.sync_copy(da_sp.at[pl.ds(base, rows)],
                        out_h.at[c, 0, pl.ds(base, rows)])
        pltpu.sync_copy(db_sp.at[pl.ds(base, rows)],
                        out_h.at[c, 1, pl.ds(base, rows)])

    return pl.kernel(
        body,
        out_type=jax.ShapeDtypeStruct((NC, 2, n_pad, DEGW), jnp.float32),
        mesh=mesh,
        scratch_types=[
            pltpu.VMEM((nch, CH), jnp.int32),
            pltpu.VMEM((CH, DEGW), jnp.float32),
            pltpu.VMEM_SHARED((n_pad, DEGW), jnp.float32),
            pltpu.VMEM_SHARED((n_pad, DEGW), jnp.float32),
        ],
        compiler_params=pltpu.CompilerParams(use_tc_tiling_on_sc=False),
    )


@functools.cache
def _build_scatter_kernel(n_pad, nch):
    rows = n_pad // NS
    mesh = plsc.VectorSubcoreMesh(core_axis_name="c", subcore_axis_name="s")

    nu = nch // UNIT

    def body(y0_h, y1_h, row_h, col_h, zer_h, out_h,
             rowv, colv, buf_a, buf_b, agg_sp,
             sem_a, sem_b, sem_sa, sem_sb):
        c = lax.axis_index("c")
        s = lax.axis_index("s")
        base = s * rows
        pltpu.sync_copy(zer_h, agg_sp.at[pl.ds(base, rows)])
        pltpu.sync_copy(row_h.at[s], rowv)
        pltpu.sync_copy(col_h.at[s], colv)
        plsc.subcore_barrier()

        def run(y_h):
            def gath(j0, buf, sem):
                for k in range(UNIT):
                    pltpu.async_copy(y_h.at[rowv.at[j0 + k]],
                                     buf.at[pl.ds(k * CH, CH)], sem)

            def gwait(j0, buf, sem):
                for k in range(UNIT):
                    pltpu.make_async_copy(y_h.at[rowv.at[j0 + k]],
                                          buf.at[pl.ds(k * CH, CH)],
                                          sem).wait()

            def scat(j0, buf, sem):
                del sem
                for k in range(UNIT):
                    pltpu.sync_copy(buf.at[pl.ds(k * CH, CH)],
                                    agg_sp.at[colv.at[j0 + k]], add=True)

            gath(0, buf_a, sem_a)

            @pl.loop(0, nu // 2)
            def _(t):
                j0 = t * (2 * UNIT)
                gath(j0 + UNIT, buf_b, sem_b)
                gwait(j0, buf_a, sem_a)
                scat(j0, buf_a, sem_sa)

                @pl.when(t < nu // 2 - 1)
                def _():
                    gath(j0 + 2 * UNIT, buf_a, sem_a)

                gwait(j0 + UNIT, buf_b, sem_b)
                scat(j0 + UNIT, buf_b, sem_sb)

        @pl.when(c == 0)
        def _():
            run(y0_h)

        @pl.when(c == 1)
        def _():
            run(y1_h)

        plsc.subcore_barrier()
        pltpu.sync_copy(agg_sp.at[pl.ds(base, rows)],
                        out_h.at[c, pl.ds(base, rows)])

    return pl.kernel(
        body,
        out_type=jax.ShapeDtypeStruct((NC, n_pad, HD), jnp.float32),
        mesh=mesh,
        scratch_types=[
            pltpu.VMEM((nch, CH), jnp.int32),
            pltpu.VMEM((nch, CH), jnp.int32),
            pltpu.VMEM((UNIT * CH, HD), jnp.float32),
            pltpu.VMEM((UNIT * CH, HD), jnp.float32),
            pltpu.VMEM_SHARED((n_pad, HD), jnp.float32),
            pltpu.SemaphoreType.DMA,
            pltpu.SemaphoreType.DMA,
            pltpu.SemaphoreType.DMA,
            pltpu.SemaphoreType.DMA,
        ],
        compiler_params=pltpu.CompilerParams(use_tc_tiling_on_sc=False),
    )


def _dot(a, b):
    return jnp.dot(a, b, preferred_element_type=jnp.float32)


def _deg_col(dp):
    return dp[0, :, 0:1] + dp[1, :, 0:1] + 1.0


def _assemble(ap):
    return jnp.concatenate([ap[0], ap[1]], axis=-1)


def _enc_body(x_ref, w1, b1, w2, b2, o_ref):
    h = _leaky(_dot(x_ref[...], w1[...]) + b1[...])
    o_ref[...] = _leaky(_dot(h, w2[...]) + b2[...])


def _emit_pre(x1, root, dp, y0_ref, y1_ref, s_ref):
    deg = _deg_col(dp)
    y = jnp.maximum(x1, 0.0) / jnp.sqrt(deg)
    y0_ref[...] = y[:, :HD]
    y1_ref[...] = y[:, HD:]
    s_ref[...] = jnp.maximum(x1 + root, 0.0) / deg


def _pre_body(h_ref, w, b, root, dp, y0_ref, y1_ref, s_ref):
    x1 = _dot(h_ref[...], w[...]) + b[...]
    _emit_pre(x1, root[...], dp[...], y0_ref, y1_ref, s_ref)


def _mid_body(ap, dpp, sp, w, b, root, dpn, y0_ref, y1_ref, s_ref):
    t = _assemble(ap[...]) / jnp.sqrt(_deg_col(dpp[...])) + sp[...]
    x1 = _dot(t, w[...]) + b[...]
    _emit_pre(x1, root[...], dpn[...], y0_ref, y1_ref, s_ref)


def _layernorm(t, g, bb):
    mu = jnp.mean(t, axis=-1, keepdims=True)
    var = jnp.mean((t - mu) ** 2, axis=-1, keepdims=True)
    return (t - mu) / jnp.sqrt(var + 1e-5) * g + bb


def _lep_body(ap, dpp, sp, g, bb, hin_ref, w, b, root, dpn,
              h1_ref, y0_ref, y1_ref, s_ref):
    t = _assemble(ap[...]) / jnp.sqrt(_deg_col(dpp[...])) + sp[...]
    h1 = _leaky(_layernorm(t, g[...], bb[...])) + hin_ref[...]
    h1_ref[...] = h1
    x1 = _dot(h1, w[...]) + b[...]
    _emit_pre(x1, root[...], dpn[...], y0_ref, y1_ref, s_ref)


def _fin_body(ap, dpp, sp, g, bb, hin_ref, o_ref):
    t = _assemble(ap[...]) / jnp.sqrt(_deg_col(dpp[...])) + sp[...]
    o_ref[...] = _leaky(_layernorm(t, g[...], bb[...])) + hin_ref[...]


def _row_spec(d):
    return pl.BlockSpec((BLK, d), lambda i: (i, 0))


def _full_spec(shape):
    nd = len(shape)
    return pl.BlockSpec(shape, lambda i, _nd=nd: (0,) * _nd)


def _agg_spec():
    return pl.BlockSpec((NC, BLK, HD), lambda i: (0, i, 0))


def _dp_spec():
    return pl.BlockSpec((NC, BLK, DEGW), lambda i: (0, i, 0))


@functools.cache
def _build_tc_kernels(n_pad, d):
    nb = n_pad // BLK
    f32 = jnp.float32
    rs, asx, dps = _row_spec(d), _agg_spec(), _dp_spec()
    hs = _row_spec(HD)
    out1 = jax.ShapeDtypeStruct((n_pad, d), f32)
    outh = jax.ShapeDtypeStruct((n_pad, HD), f32)

    enc = pl.pallas_call(
        _enc_body,
        grid=(nb,),
        in_specs=[rs, _full_spec((d, 2 * d)), _full_spec((1, 2 * d)),
                  _full_spec((2 * d, d)), _full_spec((1, d))],
        out_specs=rs,
        out_shape=out1,
    )
    pre = pl.pallas_call(
        _pre_body,
        grid=(nb,),
        in_specs=[rs, _full_spec((d, d)), _full_spec((1, d)),
                  _full_spec((1, d)), dps],
        out_specs=(hs, hs, rs),
        out_shape=(outh, outh, out1),
    )
    mid = pl.pallas_call(
        _mid_body,
        grid=(nb,),
        in_specs=[asx, dps, rs, _full_spec((d, d)), _full_spec((1, d)),
                  _full_spec((1, d)), dps],
        out_specs=(hs, hs, rs),
        out_shape=(outh, outh, out1),
    )
    lep = pl.pallas_call(
        _lep_body,
        grid=(nb,),
        in_specs=[asx, dps, rs, _full_spec((1, d)), _full_spec((1, d)), rs,
                  _full_spec((d, d)), _full_spec((1, d)), _full_spec((1, d)),
                  dps],
        out_specs=(rs, hs, hs, rs),
        out_shape=(out1, outh, outh, out1),
    )
    fin = pl.pallas_call(
        _fin_body,
        grid=(nb,),
        in_specs=[asx, dps, rs, _full_spec((1, d)), _full_spec((1, d)), rs],
        out_specs=rs,
        out_shape=out1,
    )
    return enc, pre, mid, lep, fin


def _chunk_idx(ix, fill, nworkers, nch):
    e = ix.shape[0]
    ep = nworkers * nch * CH
    pad = jnp.full((ep - e,), fill, jnp.int32)
    return jnp.concatenate([ix.astype(jnp.int32), pad]).reshape(
        nworkers, nch, CH)


def kernel(x, edge_index_node_net, edge_index_net_node,
           enc_w1, enc_b1, enc_w2, enc_b2,
           cw0, cb0, cr0, rw0, rb0, rr0, ln_g0, ln_b0,
           cw1, cb1, cr1, rw1, rb1, rr1, ln_g1, ln_b1):
    n, d = x.shape
    e = edge_index_node_net.shape[1]
    n_pad = -(-max(n + 1, BLK) // (BLK * 2)) * (BLK * 2)
    nch_deg = -(-e // (NW * CH))
    nch_deg += nch_deg % 2
    nch = -(-e // (NS * CH))
    nch = -(-nch // (2 * UNIT)) * (2 * UNIT)

    rad = _chunk_idx(edge_index_node_net[0], n, NW, nch_deg)
    rbd = _chunk_idx(edge_index_net_node[0], n, NW, nch_deg)
    row_a = _chunk_idx(edge_index_node_net[0], n, NS, nch)
    col_a = _chunk_idx(edge_index_node_net[1], n, NS, nch)
    row_b = _chunk_idx(edge_index_net_node[0], n, NS, nch)
    col_b = _chunk_idx(edge_index_net_node[1], n, NS, nch)
    x_pad = jnp.pad(x, ((0, n_pad - n), (0, 0)))

    ones_h = jnp.ones((CH, DEGW), jnp.float32)
    zer_deg = jnp.zeros((n_pad // NS, DEGW), jnp.float32)
    zer_agg = jnp.zeros((n_pad // NS, HD), jnp.float32)

    deg_k = _build_deg_kernel(n_pad, nch_deg)
    scat_k = _build_scatter_kernel(n_pad, nch)
    enc, pre, mid, lep, fin = _build_tc_kernels(n_pad, d)

    r1 = lambda v: v.reshape(1, -1)

    deg_parts = deg_k(rad, rbd, ones_h, zer_deg)
    dp_a = deg_parts[:, 0]
    dp_b = deg_parts[:, 1]

    h0 = enc(x_pad, enc_w1, r1(enc_b1), enc_w2, r1(enc_b2))
    y0, y1, sf = pre(h0, cw0, r1(cb0), r1(cr0), dp_a)
    ag = scat_k(y0, y1, row_a, col_a, zer_agg)
    y0, y1, sf = mid(ag, dp_a, sf, rw0, r1(rb0), r1(rr0), dp_b)
    ag = scat_k(y0, y1, row_b, col_b, zer_agg)
    h1, y0, y1, sf = lep(ag, dp_b, sf, r1(ln_g0), r1(ln_b0), h0,
                         cw1, r1(cb1), r1(cr1), dp_a)
    ag = scat_k(y0, y1, row_a, col_a, zer_agg)
    y0, y1, sf = mid(ag, dp_a, sf, rw1, r1(rb1), r1(rr1), dp_b)
    ag = scat_k(y0, y1, row_b, col_b, zer_agg)
    out = fin(ag, dp_b, sf, r1(ln_g1), r1(ln_b1), h1)
    return out[:n]


# speedup vs baseline: 1.3503x; 1.3503x over previous
import functools

import jax
import jax.numpy as jnp
from jax import lax
from jax.experimental import pallas as pl
from jax.experimental.pallas import tpu as pltpu
from jax.experimental.pallas import tpu_sc as plsc

NS = 16
NC = 2
NW = NC * NS
CH = 128
BLK = 512
DEGW = 16
HD = 64


def _leaky(v):
    return jnp.where(v > 0, v, 0.1 * v)


@functools.cache
def _build_deg_kernel(n_pad, nch):
    rows = n_pad // NS
    mesh = plsc.VectorSubcoreMesh(core_axis_name="c", subcore_axis_name="s")

    def body(row_a, row_b, ones_h, zer_h, out_h, idx_v, ones_v, da_sp, db_sp):
        c = lax.axis_index("c")
        s = lax.axis_index("s")
        wid = c * NS + s
        base = s * rows
        pltpu.sync_copy(zer_h, da_sp.at[pl.ds(base, rows)])
        pltpu.sync_copy(zer_h, db_sp.at[pl.ds(base, rows)])
        pltpu.sync_copy(ones_h, ones_v)
        pltpu.sync_copy(row_a.at[wid], idx_v)
        plsc.subcore_barrier()

        @pl.loop(0, nch)
        def _(j):
            pltpu.sync_copy(ones_v, da_sp.at[idx_v.at[j]], add=True)

        pltpu.sync_copy(row_b.at[wid], idx_v)

        @pl.loop(0, nch)
        def _(j):
            pltpu.sync_copy(ones_v, db_sp.at[idx_v.at[j]], add=True)

        plsc.subcore_barrier()
        pltpu.sync_copy(da_sp.at[pl.ds(base, rows)],
                        out_h.at[c, 0, pl.ds(base, rows)])
        pltpu.sync_copy(db_sp.at[pl.ds(base, rows)],
                        out_h.at[c, 1, pl.ds(base, rows)])

    return pl.kernel(
        body,
        out_type=jax.ShapeDtypeStruct((NC, 2, n_pad, DEGW), jnp.float32),
        mesh=mesh,
        scratch_types=[
            pltpu.VMEM((nch, CH), jnp.int32),
            pltpu.VMEM((CH, DEGW), jnp.float32),
            pltpu.VMEM_SHARED((n_pad, DEGW), jnp.float32),
            pltpu.VMEM_SHARED((n_pad, DEGW), jnp.float32),
        ],
        compiler_params=pltpu.CompilerParams(use_tc_tiling_on_sc=False),
    )


@functools.cache
def _build_scatter_kernel(n_pad, nch):
    rows = n_pad // NS
    mesh = plsc.VectorSubcoreMesh(core_axis_name="c", subcore_axis_name="s")

    def body(y0_h, y1_h, row_h, col_h, zer_h, out_h,
             rowv, colv, buf_a, buf_b, agg_sp, sem_a, sem_b):
        c = lax.axis_index("c")
        s = lax.axis_index("s")
        base = s * rows
        pltpu.sync_copy(zer_h, agg_sp.at[pl.ds(base, rows)])
        pltpu.sync_copy(row_h.at[s], rowv)
        pltpu.sync_copy(col_h.at[s], colv)
        plsc.subcore_barrier()

        def run(y_h):
            pltpu.async_copy(y_h.at[rowv.at[0]], buf_a, sem_a)

            @pl.loop(0, nch // 2)
            def _(t):
                j0 = t * 2
                pltpu.async_copy(y_h.at[rowv.at[j0 + 1]], buf_b, sem_b)
                pltpu.make_async_copy(y_h.at[rowv.at[j0]], buf_a, sem_a).wait()
                pltpu.sync_copy(buf_a, agg_sp.at[colv.at[j0]], add=True)

                @pl.when(t < nch // 2 - 1)
                def _():
                    pltpu.async_copy(y_h.at[rowv.at[j0 + 2]], buf_a, sem_a)

                pltpu.make_async_copy(
                    y_h.at[rowv.at[j0 + 1]], buf_b, sem_b).wait()
                pltpu.sync_copy(buf_b, agg_sp.at[colv.at[j0 + 1]], add=True)

        @pl.when(c == 0)
        def _():
            run(y0_h)

        @pl.when(c == 1)
        def _():
            run(y1_h)

        plsc.subcore_barrier()
        pltpu.sync_copy(agg_sp.at[pl.ds(base, rows)],
                        out_h.at[c, pl.ds(base, rows)])

    return pl.kernel(
        body,
        out_type=jax.ShapeDtypeStruct((NC, n_pad, HD), jnp.float32),
        mesh=mesh,
        scratch_types=[
            pltpu.VMEM((nch, CH), jnp.int32),
            pltpu.VMEM((nch, CH), jnp.int32),
            pltpu.VMEM((CH, HD), jnp.float32),
            pltpu.VMEM((CH, HD), jnp.float32),
            pltpu.VMEM_SHARED((n_pad, HD), jnp.float32),
            pltpu.SemaphoreType.DMA,
            pltpu.SemaphoreType.DMA,
        ],
        compiler_params=pltpu.CompilerParams(use_tc_tiling_on_sc=False),
    )


def _dot(a, b):
    return jnp.dot(a, b, preferred_element_type=jnp.float32)


def _deg_col(dp):
    return dp[0, :, 0:1] + dp[1, :, 0:1] + 1.0


def _assemble(ap):
    return jnp.concatenate([ap[0], ap[1]], axis=-1)


def _enc_body(x_ref, w1, b1, w2, b2, o_ref):
    h = _leaky(_dot(x_ref[...], w1[...]) + b1[...])
    o_ref[...] = _leaky(_dot(h, w2[...]) + b2[...])


def _emit_pre(x1, root, dp, y0_ref, y1_ref, s_ref):
    deg = _deg_col(dp)
    y = jnp.maximum(x1, 0.0) / jnp.sqrt(deg)
    y0_ref[...] = y[:, :HD]
    y1_ref[...] = y[:, HD:]
    s_ref[...] = jnp.maximum(x1 + root, 0.0) / deg


def _pre_body(h_ref, w, b, root, dp, y0_ref, y1_ref, s_ref):
    x1 = _dot(h_ref[...], w[...]) + b[...]
    _emit_pre(x1, root[...], dp[...], y0_ref, y1_ref, s_ref)


def _mid_body(ap, dpp, sp, w, b, root, dpn, y0_ref, y1_ref, s_ref):
    t = _assemble(ap[...]) / jnp.sqrt(_deg_col(dpp[...])) + sp[...]
    x1 = _dot(t, w[...]) + b[...]
    _emit_pre(x1, root[...], dpn[...], y0_ref, y1_ref, s_ref)


def _layernorm(t, g, bb):
    mu = jnp.mean(t, axis=-1, keepdims=True)
    var = jnp.mean((t - mu) ** 2, axis=-1, keepdims=True)
    return (t - mu) / jnp.sqrt(var + 1e-5) * g + bb


def _lep_body(ap, dpp, sp, g, bb, hin_ref, w, b, root, dpn,
              h1_ref, y0_ref, y1_ref, s_ref):
    t = _assemble(ap[...]) / jnp.sqrt(_deg_col(dpp[...])) + sp[...]
    h1 = _leaky(_layernorm(t, g[...], bb[...])) + hin_ref[...]
    h1_ref[...] = h1
    x1 = _dot(h1, w[...]) + b[...]
    _emit_pre(x1, root[...], dpn[...], y0_ref, y1_ref, s_ref)


def _fin_body(ap, dpp, sp, g, bb, hin_ref, o_ref):
    t = _assemble(ap[...]) / jnp.sqrt(_deg_col(dpp[...])) + sp[...]
    o_ref[...] = _leaky(_layernorm(t, g[...], bb[...])) + hin_ref[...]


def _row_spec(d):
    return pl.BlockSpec((BLK, d), lambda i: (i, 0))


def _full_spec(shape):
    nd = len(shape)
    return pl.BlockSpec(shape, lambda i, _nd=nd: (0,) * _nd)


def _agg_spec():
    return pl.BlockSpec((NC, BLK, HD), lambda i: (0, i, 0))


def _dp_spec():
    return pl.BlockSpec((NC, BLK, DEGW), lambda i: (0, i, 0))


@functools.cache
def _build_tc_kernels(n_pad, d):
    nb = n_pad // BLK
    f32 = jnp.float32
    rs, asx, dps = _row_spec(d), _agg_spec(), _dp_spec()
    hs = _row_spec(HD)
    out1 = jax.ShapeDtypeStruct((n_pad, d), f32)
    outh = jax.ShapeDtypeStruct((n_pad, HD), f32)

    enc = pl.pallas_call(
        _enc_body,
        grid=(nb,),
        in_specs=[rs, _full_spec((d, 2 * d)), _full_spec((1, 2 * d)),
                  _full_spec((2 * d, d)), _full_spec((1, d))],
        out_specs=rs,
        out_shape=out1,
    )
    pre = pl.pallas_call(
        _pre_body,
        grid=(nb,),
        in_specs=[rs, _full_spec((d, d)), _full_spec((1, d)),
                  _full_spec((1, d)), dps],
        out_specs=(hs, hs, rs),
        out_shape=(outh, outh, out1),
    )
    mid = pl.pallas_call(
        _mid_body,
        grid=(nb,),
        in_specs=[asx, dps, rs, _full_spec((d, d)), _full_spec((1, d)),
                  _full_spec((1, d)), dps],
        out_specs=(hs, hs, rs),
        out_shape=(outh, outh, out1),
    )
    lep = pl.pallas_call(
        _lep_body,
        grid=(nb,),
        in_specs=[asx, dps, rs, _full_spec((1, d)), _full_spec((1, d)), rs,
                  _full_spec((d, d)), _full_spec((1, d)), _full_spec((1, d)),
                  dps],
        out_specs=(rs, hs, hs, rs),
        out_shape=(out1, outh, outh, out1),
    )
    fin = pl.pallas_call(
        _fin_body,
        grid=(nb,),
        in_specs=[asx, dps, rs, _full_spec((1, d)), _full_spec((1, d)), rs],
        out_specs=rs,
        out_shape=out1,
    )
    return enc, pre, mid, lep, fin


def _chunk_idx(ix, fill, nworkers, nch):
    e = ix.shape[0]
    ep = nworkers * nch * CH
    pad = jnp.full((ep - e,), fill, jnp.int32)
    return jnp.concatenate([ix.astype(jnp.int32), pad]).reshape(
        nworkers, nch, CH)


def kernel(x, edge_index_node_net, edge_index_net_node,
           enc_w1, enc_b1, enc_w2, enc_b2,
           cw0, cb0, cr0, rw0, rb0, rr0, ln_g0, ln_b0,
           cw1, cb1, cr1, rw1, rb1, rr1, ln_g1, ln_b1):
    n, d = x.shape
    e = edge_index_node_net.shape[1]
    n_pad = -(-max(n + 1, BLK) // (BLK * 2)) * (BLK * 2)
    nch_deg = -(-e // (NW * CH))
    nch_deg += nch_deg % 2
    nch = -(-e // (NS * CH))
    nch += nch % 2

    rad = _chunk_idx(edge_index_node_net[0], n, NW, nch_deg)
    rbd = _chunk_idx(edge_index_net_node[0], n, NW, nch_deg)
    row_a = _chunk_idx(edge_index_node_net[0], n, NS, nch)
    col_a = _chunk_idx(edge_index_node_net[1], n, NS, nch)
    row_b = _chunk_idx(edge_index_net_node[0], n, NS, nch)
    col_b = _chunk_idx(edge_index_net_node[1], n, NS, nch)
    x_pad = jnp.pad(x, ((0, n_pad - n), (0, 0)))

    ones_h = jnp.ones((CH, DEGW), jnp.float32)
    zer_deg = jnp.zeros((n_pad // NS, DEGW), jnp.float32)
    zer_agg = jnp.zeros((n_pad // NS, HD), jnp.float32)

    deg_k = _build_deg_kernel(n_pad, nch_deg)
    scat_k = _build_scatter_kernel(n_pad, nch)
    enc, pre, mid, lep, fin = _build_tc_kernels(n_pad, d)

    r1 = lambda v: v.reshape(1, -1)

    deg_parts = deg_k(rad, rbd, ones_h, zer_deg)
    dp_a = deg_parts[:, 0]
    dp_b = deg_parts[:, 1]

    h0 = enc(x_pad, enc_w1, r1(enc_b1), enc_w2, r1(enc_b2))
    y0, y1, sf = pre(h0, cw0, r1(cb0), r1(cr0), dp_a)
    ag = scat_k(y0, y1, row_a, col_a, zer_agg)
    y0, y1, sf = mid(ag, dp_a, sf, rw0, r1(rb0), r1(rr0), dp_b)
    ag = scat_k(y0, y1, row_b, col_b, zer_agg)
    h1, y0, y1, sf = lep(ag, dp_b, sf, r1(ln_g0), r1(ln_b0), h0,
                         cw1, r1(cb1), r1(cr1), dp_a)
    ag = scat_k(y0, y1, row_a, col_a, zer_agg)
    y0, y1, sf = mid(ag, dp_a, sf, rw1, r1(rb1), r1(rr1), dp_b)
    ag = scat_k(y0, y1, row_b, col_b, zer_agg)
    out = fin(ag, dp_b, sf, r1(ln_g1), r1(ln_b1), h1)
    return out[:n]
